# baseline (speedup 1.0000x reference)
import jax
import jax.numpy as jnp
from jax.experimental import pallas as pl

TOKEN_BLOCK = 1024


def _moe_kernel(x_ref, gate_w_ref, gate_b_ref, ew_ref, eb_ref, out_ref):
    x = x_ref[...]
    logits = jnp.dot(x, gate_w_ref[...], preferred_element_type=jnp.float32)
    logits = logits + gate_b_ref[...]
    m = jnp.max(logits, axis=-1, keepdims=True)
    ex = jnp.exp(logits - m)
    w = ex / jnp.sum(ex, axis=-1, keepdims=True)

    bias = jnp.dot(w, eb_ref[...], preferred_element_type=jnp.float32)

    x_bf = x.astype(jnp.bfloat16)
    w_bf = w.astype(jnp.bfloat16)
    num_expert = eb_ref.shape[0]
    xw = jnp.concatenate(
        [x_bf * w_bf[:, e:e + 1] for e in range(num_expert)], axis=1)
    out_ref[...] = bias + jnp.dot(
        xw, ew_ref[...], preferred_element_type=jnp.float32)


def kernel(X, gate_W, gate_b, expert_W, expert_b):
    tokens, f_in = X.shape
    num_expert, _, f_out = expert_W.shape
    ew_bf = expert_W.astype(jnp.bfloat16).reshape(num_expert * f_in, f_out)
    gate_b2 = gate_b.reshape(1, num_expert)

    grid = (tokens // TOKEN_BLOCK,)
    return pl.pallas_call(
        _moe_kernel,
        grid=grid,
        in_specs=[
            pl.BlockSpec((TOKEN_BLOCK, f_in), lambda i: (i, 0)),
            pl.BlockSpec((f_in, num_expert), lambda i: (0, 0)),
            pl.BlockSpec((1, num_expert), lambda i: (0, 0)),
            pl.BlockSpec((num_expert * f_in, f_out), lambda i: (0, 0)),
            pl.BlockSpec((num_expert, f_out), lambda i: (0, 0)),
        ],
        out_specs=pl.BlockSpec((TOKEN_BLOCK, f_out), lambda i: (i, 0)),
        out_shape=jax.ShapeDtypeStruct((tokens, f_out), jnp.float32),
    )(X, gate_W, gate_b2, ew_bf, expert_b)


# speedup vs baseline: 1.0731x; 1.0731x over previous
import jax
import jax.numpy as jnp
from jax.experimental import pallas as pl

TOKEN_BLOCK = 1024


def _moe_kernel(x_ref, gate_w_ref, gate_b_ref, ew_ref, eb_ref, out_ref):
    x = x_ref[...]
    logits = jnp.dot(x, gate_w_ref[...], preferred_element_type=jnp.float32)
    logits = logits + gate_b_ref[...]
    m = jnp.max(logits, axis=-1, keepdims=True)
    ex = jnp.exp(logits - m)
    w = ex / jnp.sum(ex, axis=-1, keepdims=True)

    bias = jnp.dot(w, eb_ref[...], preferred_element_type=jnp.float32)

    x_bf = x.astype(jnp.bfloat16)
    num_expert = eb_ref.shape[0]
    f_in = x.shape[1]
    acc = bias
    for e in range(num_expert):
        pe = jnp.dot(x_bf, ew_ref[e * f_in:(e + 1) * f_in, :],
                     preferred_element_type=jnp.float32)
        acc = acc + w[:, e:e + 1] * pe
    out_ref[...] = acc


def kernel(X, gate_W, gate_b, expert_W, expert_b):
    tokens, f_in = X.shape
    num_expert, _, f_out = expert_W.shape
    ew_bf = expert_W.astype(jnp.bfloat16).reshape(num_expert * f_in, f_out)
    gate_b2 = gate_b.reshape(1, num_expert)

    grid = (tokens // TOKEN_BLOCK,)
    return pl.pallas_call(
        _moe_kernel,
        grid=grid,
        in_specs=[
            pl.BlockSpec((TOKEN_BLOCK, f_in), lambda i: (i, 0)),
            pl.BlockSpec((f_in, num_expert), lambda i: (0, 0)),
            pl.BlockSpec((1, num_expert), lambda i: (0, 0)),
            pl.BlockSpec((num_expert * f_in, f_out), lambda i: (0, 0)),
            pl.BlockSpec((num_expert, f_out), lambda i: (0, 0)),
        ],
        out_specs=pl.BlockSpec((TOKEN_BLOCK, f_out), lambda i: (i, 0)),
        out_shape=jax.ShapeDtypeStruct((tokens, f_out), jnp.float32),
    )(X, gate_W, gate_b2, ew_bf, expert_b)
